# baseline (device time: 15953 ns/iter reference)
import jax
import jax.numpy as jnp
from jax import lax
from jax.experimental import pallas as pl
from jax.experimental.pallas import tpu as pltpu

CHUNK_ROWS = (8, 8, 16, 24, 32, 48, 56, 64)
S = len(CHUNK_ROWS)


def kernel(x):
    _, m, n = x.shape
    half = n // 2
    mrows = m // 2
    assert sum(CHUNK_ROWS) == mrows
    offs = [sum(CHUNK_ROWS[:i]) for i in range(S)]

    def body(x_ref, out_ref, recv1, sems1_send, sems1_recv, sems2_send, sems2_recv):
        my_x = lax.axis_index("x")
        my_y = lax.axis_index("y")
        ypeer = (my_x, 1 - my_y)
        xpeer = (1 - my_x, my_y)

        my_row0 = my_x * mrows
        other_row0 = (1 - my_x) * mrows
        my_col0 = my_y * half
        peer_col0 = (1 - my_y) * half

        barrier_sem = pltpu.get_barrier_semaphore()
        for nbr in (ypeer, xpeer):
            pl.semaphore_signal(
                barrier_sem, inc=1, device_id=nbr,
                device_id_type=pl.DeviceIdType.MESH,
            )
        pl.semaphore_wait(barrier_sem, 2)

        d1 = []
        for s in range(S):
            o, r = offs[s], CHUNK_ROWS[s]
            d = pltpu.make_async_remote_copy(
                src_ref=x_ref.at[0, pl.ds(my_row0 + o, r), pl.ds(peer_col0, half)],
                dst_ref=recv1.at[pl.ds(o, r), :],
                send_sem=sems1_send.at[s],
                recv_sem=sems1_recv.at[s],
                device_id=ypeer,
                device_id_type=pl.DeviceIdType.MESH,
            )
            d.start()
            d1.append(d)

        d2 = []
        for s in range(S):
            o, r = offs[s], CHUNK_ROWS[s]
            rows = pl.ds(my_row0 + o, r)
            d1[s].wait_recv()
            out_ref[rows, :] = (
                x_ref[0, pl.ds(my_row0 + o, r), pl.ds(my_col0, half)]
                + recv1[pl.ds(o, r), :]
            )
            d = pltpu.make_async_remote_copy(
                src_ref=out_ref.at[rows, :],
                dst_ref=out_ref.at[rows, :],
                send_sem=sems2_send.at[s],
                recv_sem=sems2_recv.at[s],
                device_id=xpeer,
                device_id_type=pl.DeviceIdType.MESH,
            )
            d.start()
            d2.append(d)

        for s in range(S):
            rows_in = pl.ds(other_row0 + offs[s], CHUNK_ROWS[s])
            recv_desc = pltpu.make_async_remote_copy(
                src_ref=out_ref.at[rows_in, :],
                dst_ref=out_ref.at[rows_in, :],
                send_sem=sems2_send.at[s],
                recv_sem=sems2_recv.at[s],
                device_id=xpeer,
                device_id_type=pl.DeviceIdType.MESH,
            )
            recv_desc.wait_recv()

        for s in range(S):
            d1[s].wait_send()
            d2[s].wait_send()

    return pl.pallas_call(
        body,
        out_shape=jax.ShapeDtypeStruct((m, half), x.dtype),
        in_specs=[pl.BlockSpec(memory_space=pltpu.VMEM)],
        out_specs=pl.BlockSpec(memory_space=pltpu.VMEM),
        scratch_shapes=[
            pltpu.VMEM((mrows, half), x.dtype),
            pltpu.SemaphoreType.DMA((S,)),
            pltpu.SemaphoreType.DMA((S,)),
            pltpu.SemaphoreType.DMA((S,)),
            pltpu.SemaphoreType.DMA((S,)),
        ],
        compiler_params=pltpu.CompilerParams(collective_id=0),
    )(x)


# device time: 15211 ns/iter; 1.0488x vs baseline; 1.0488x over previous
import jax
import jax.numpy as jnp
from jax import lax
from jax.experimental import pallas as pl
from jax.experimental.pallas import tpu as pltpu

CHUNK_ROWS = (16,) * 16
S = len(CHUNK_ROWS)


def kernel(x):
    _, m, n = x.shape
    half = n // 2
    mrows = m // 2
    assert sum(CHUNK_ROWS) == mrows
    offs = [sum(CHUNK_ROWS[:i]) for i in range(S)]

    def body(x_ref, out_ref, recv1, sems1_send, sems1_recv, sems2_send, sems2_recv):
        my_x = lax.axis_index("x")
        my_y = lax.axis_index("y")
        ypeer = (my_x, 1 - my_y)
        xpeer = (1 - my_x, my_y)

        my_row0 = my_x * mrows
        other_row0 = (1 - my_x) * mrows
        my_col0 = my_y * half
        peer_col0 = (1 - my_y) * half

        barrier_sem = pltpu.get_barrier_semaphore()
        for nbr in (ypeer, xpeer):
            pl.semaphore_signal(
                barrier_sem, inc=1, device_id=nbr,
                device_id_type=pl.DeviceIdType.MESH,
            )
        pl.semaphore_wait(barrier_sem, 2)

        d1 = []
        for s in range(S):
            o, r = offs[s], CHUNK_ROWS[s]
            d = pltpu.make_async_remote_copy(
                src_ref=x_ref.at[0, pl.ds(my_row0 + o, r), pl.ds(peer_col0, half)],
                dst_ref=recv1.at[pl.ds(o, r), :],
                send_sem=sems1_send.at[s],
                recv_sem=sems1_recv.at[s],
                device_id=ypeer,
                device_id_type=pl.DeviceIdType.MESH,
            )
            d.start()
            d1.append(d)

        d2 = []
        for s in range(S):
            o, r = offs[s], CHUNK_ROWS[s]
            rows = pl.ds(my_row0 + o, r)
            d1[s].wait_recv()
            out_ref[rows, :] = (
                x_ref[0, pl.ds(my_row0 + o, r), pl.ds(my_col0, half)]
                + recv1[pl.ds(o, r), :]
            )
            d = pltpu.make_async_remote_copy(
                src_ref=out_ref.at[rows, :],
                dst_ref=out_ref.at[rows, :],
                send_sem=sems2_send.at[s],
                recv_sem=sems2_recv.at[s],
                device_id=xpeer,
                device_id_type=pl.DeviceIdType.MESH,
            )
            d.start()
            d2.append(d)

        for s in range(S):
            rows_in = pl.ds(other_row0 + offs[s], CHUNK_ROWS[s])
            recv_desc = pltpu.make_async_remote_copy(
                src_ref=out_ref.at[rows_in, :],
                dst_ref=out_ref.at[rows_in, :],
                send_sem=sems2_send.at[s],
                recv_sem=sems2_recv.at[s],
                device_id=xpeer,
                device_id_type=pl.DeviceIdType.MESH,
            )
            recv_desc.wait_recv()

        for s in range(S):
            d1[s].wait_send()
            d2[s].wait_send()

    return pl.pallas_call(
        body,
        out_shape=jax.ShapeDtypeStruct((m, half), x.dtype),
        in_specs=[pl.BlockSpec(memory_space=pltpu.VMEM)],
        out_specs=pl.BlockSpec(memory_space=pltpu.VMEM),
        scratch_shapes=[
            pltpu.VMEM((mrows, half), x.dtype),
            pltpu.SemaphoreType.DMA((S,)),
            pltpu.SemaphoreType.DMA((S,)),
            pltpu.SemaphoreType.DMA((S,)),
            pltpu.SemaphoreType.DMA((S,)),
        ],
        compiler_params=pltpu.CompilerParams(collective_id=0),
    )(x)


# device time: 14675 ns/iter; 1.0871x vs baseline; 1.0365x over previous
import jax
import jax.numpy as jnp
from jax import lax
from jax.experimental import pallas as pl
from jax.experimental.pallas import tpu as pltpu

S = 16
DUP_ROWS = 48


def kernel(x):
    _, m, n = x.shape
    half = n // 2
    mrows = m // 2
    r = mrows // S
    fwd_rows = mrows - DUP_ROWS
    S2 = fwd_rows // r
    assert fwd_rows % r == 0

    def body(x_ref, out_ref, recv1, recvd, sems1_send, sems1_recv,
             sems2_send, sems2_recv, semd_send, semd_recv):
        my_x = lax.axis_index("x")
        my_y = lax.axis_index("y")
        ypeer = (my_x, 1 - my_y)
        xpeer = (1 - my_x, my_y)

        my_row0 = my_x * mrows
        other_row0 = (1 - my_x) * mrows
        dup_row0 = other_row0 + fwd_rows
        my_col0 = my_y * half
        peer_col0 = (1 - my_y) * half

        barrier_sem = pltpu.get_barrier_semaphore()
        for nbr in (ypeer, xpeer):
            pl.semaphore_signal(
                barrier_sem, inc=1, device_id=nbr,
                device_id_type=pl.DeviceIdType.MESH,
            )
        pl.semaphore_wait(barrier_sem, 2)

        d1 = []
        for s in range(S):
            d = pltpu.make_async_remote_copy(
                src_ref=x_ref.at[0, pl.ds(my_row0 + s * r, r), pl.ds(peer_col0, half)],
                dst_ref=recv1.at[pl.ds(s * r, r), :],
                send_sem=sems1_send.at[s],
                recv_sem=sems1_recv.at[s],
                device_id=ypeer,
                device_id_type=pl.DeviceIdType.MESH,
            )
            d.start()
            d1.append(d)
        dd = pltpu.make_async_remote_copy(
            src_ref=x_ref.at[0, pl.ds(dup_row0, DUP_ROWS), pl.ds(peer_col0, half)],
            dst_ref=recvd,
            send_sem=semd_send,
            recv_sem=semd_recv,
            device_id=ypeer,
            device_id_type=pl.DeviceIdType.MESH,
        )
        dd.start()

        d2 = []
        for s in range(S):
            rows = pl.ds(my_row0 + s * r, r)
            d1[s].wait_recv()
            out_ref[rows, :] = (
                x_ref[0, pl.ds(my_row0 + s * r, r), pl.ds(my_col0, half)]
                + recv1[pl.ds(s * r, r), :]
            )
            if s < S2:
                d = pltpu.make_async_remote_copy(
                    src_ref=out_ref.at[rows, :],
                    dst_ref=out_ref.at[rows, :],
                    send_sem=sems2_send.at[s],
                    recv_sem=sems2_recv.at[s],
                    device_id=xpeer,
                    device_id_type=pl.DeviceIdType.MESH,
                )
                d.start()
                d2.append(d)

        dd.wait_recv()
        out_ref[pl.ds(dup_row0, DUP_ROWS), :] = (
            x_ref[0, pl.ds(dup_row0, DUP_ROWS), pl.ds(my_col0, half)]
            + recvd[...]
        )

        for s in range(S2):
            rows_in = pl.ds(other_row0 + s * r, r)
            recv_desc = pltpu.make_async_remote_copy(
                src_ref=out_ref.at[rows_in, :],
                dst_ref=out_ref.at[rows_in, :],
                send_sem=sems2_send.at[s],
                recv_sem=sems2_recv.at[s],
                device_id=xpeer,
                device_id_type=pl.DeviceIdType.MESH,
            )
            recv_desc.wait_recv()

        for s in range(S):
            d1[s].wait_send()
        for d in d2:
            d.wait_send()
        dd.wait_send()

    return pl.pallas_call(
        body,
        out_shape=jax.ShapeDtypeStruct((m, half), x.dtype),
        in_specs=[pl.BlockSpec(memory_space=pltpu.VMEM)],
        out_specs=pl.BlockSpec(memory_space=pltpu.VMEM),
        scratch_shapes=[
            pltpu.VMEM((mrows, half), x.dtype),
            pltpu.VMEM((DUP_ROWS, half), x.dtype),
            pltpu.SemaphoreType.DMA((S,)),
            pltpu.SemaphoreType.DMA((S,)),
            pltpu.SemaphoreType.DMA((S,)),
            pltpu.SemaphoreType.DMA((S,)),
            pltpu.SemaphoreType.DMA,
            pltpu.SemaphoreType.DMA,
        ],
        compiler_params=pltpu.CompilerParams(collective_id=0),
    )(x)
